# baseline (speedup 1.0000x reference)
import functools

import jax
import jax.numpy as jnp
from jax import lax
from jax.experimental import pallas as pl
from jax.experimental.pallas import tpu as pltpu
from jax.experimental.pallas import tpu_sc as plsc

D = 1024
HALF = 512
N_TOKENS = 8192
NW = 32
TPW = N_TOKENS // NW
CHUNK = 32
N_CHUNKS = TPW // CHUNK
LANES = 16
SLICES = D // LANES
HSLICES = HALF // LANES


def _sc_encode(xf, gi, by, tab, wp):
    mesh = plsc.VectorSubcoreMesh(core_axis_name="c", subcore_axis_name="s")

    @functools.partial(
        pl.kernel,
        mesh=mesh,
        out_type=jax.ShapeDtypeStruct((N_TOKENS, D), jnp.float32),
        scratch_types=[
            pltpu.VMEM((HALF,), jnp.float32),
            pltpu.VMEM((CHUNK,), jnp.int32),
            pltpu.VMEM((CHUNK + LANES,), jnp.float32),
            pltpu.VMEM((CHUNK, D), jnp.float32),
            pltpu.VMEM((CHUNK, D), jnp.float32),
            pltpu.SemaphoreType.DMA,
        ],
    )
    def k(x_hbm, gi_hbm, by_hbm, tab_hbm, wp_hbm, out_hbm,
          wp_v, gic, nyc, trows, xb, sem):
        wid = lax.axis_index("s") * 2 + lax.axis_index("c")
        base = wid * TPW
        pltpu.sync_copy(wp_hbm, wp_v)
        for c in range(N_CHUNKS):
            t0 = base + c * CHUNK
            pltpu.sync_copy(gi_hbm.at[pl.ds(t0, CHUNK)], gic)
            pltpu.sync_copy(by_hbm.at[pl.ds(t0, CHUNK)], nyc.at[pl.ds(0, CHUNK)])
            pltpu.async_copy(tab_hbm.at[gic], trows, sem).wait()
            pltpu.sync_copy(x_hbm.at[pl.ds(t0, CHUNK)], xb)
            for s in range(CHUNK // LANES):
                sl = pl.ds(s * LANES, LANES)
                nyc[sl] = (nyc[sl] - 1900.0) * 0.01

            def body(i, _):
                nyw = nyc[pl.ds(i, LANES)]
                ny16 = jnp.full((LANES,), nyw[0], jnp.float32)
                for j in range(SLICES):
                    sl = pl.ds(j * LANES, LANES)
                    v = xb[i, sl] + trows[i, sl]
                    if j < HSLICES:
                        v = v + ny16 * wp_v[sl]
                    xb[i, sl] = v
                return 0

            lax.fori_loop(0, CHUNK, body, 0)
            pltpu.sync_copy(xb, out_hbm.at[pl.ds(t0, CHUNK)])

    return k(xf, gi, by, tab, wp)


def kernel(x, generation_info, birth_years, gen_table, temporal_W, temporal_b):
    B, L, d = x.shape
    xf = x.reshape(B * L, d)
    gi = generation_info.reshape(-1).astype(jnp.int32)
    by = birth_years.reshape(-1)
    bp = jnp.pad(temporal_b, (0, d - temporal_b.shape[0]))
    tab = gen_table + bp[None, :]
    wp = temporal_W[:, 0]
    out = _sc_encode(xf, gi, by, tab, wp)
    return out.reshape(B, L, d)


# speedup vs baseline: 1.2621x; 1.2621x over previous
import functools

import jax
import jax.numpy as jnp
from jax import lax
from jax.experimental import pallas as pl
from jax.experimental.pallas import tpu as pltpu
from jax.experimental.pallas import tpu_sc as plsc

D = 1024
HALF = 512
N_TOKENS = 8192
NW = 32
TPW = N_TOKENS // NW
CHUNK = 16
N_CHUNKS = TPW // CHUNK
LANES = 16
SLICES = D // LANES
HSLICES = HALF // LANES


def _sc_encode(xf, gi, by, tab, wp):
    mesh = plsc.VectorSubcoreMesh(core_axis_name="c", subcore_axis_name="s")

    @functools.partial(
        pl.kernel,
        mesh=mesh,
        out_type=jax.ShapeDtypeStruct((N_TOKENS, D), jnp.float32),
        scratch_types=[
            pltpu.VMEM((HALF,), jnp.float32),
            pltpu.VMEM((TPW,), jnp.int32),
            pltpu.VMEM((TPW + LANES,), jnp.float32),
            pltpu.VMEM((CHUNK, D), jnp.float32),
            pltpu.VMEM((CHUNK, D), jnp.float32),
            pltpu.VMEM((CHUNK, D), jnp.float32),
            pltpu.VMEM((CHUNK, D), jnp.float32),
            pltpu.SemaphoreType.DMA,
            pltpu.SemaphoreType.DMA,
            pltpu.SemaphoreType.DMA,
            pltpu.SemaphoreType.DMA,
            pltpu.SemaphoreType.DMA,
            pltpu.SemaphoreType.DMA,
        ],
    )
    def k(x_hbm, gi_hbm, by_hbm, tab_hbm, wp_hbm, out_hbm,
          wp_v, gic, nyc, xb0, xb1, tr0, tr1, sx0, sx1, sg0, sg1, so0, so1):
        wid = lax.axis_index("s") * 2 + lax.axis_index("c")
        base = wid * TPW
        pltpu.sync_copy(wp_hbm, wp_v)
        pltpu.sync_copy(gi_hbm.at[pl.ds(base, TPW)], gic)
        pltpu.sync_copy(by_hbm.at[pl.ds(base, TPW)], nyc.at[pl.ds(0, TPW)])
        for s in range(TPW // LANES):
            sl = pl.ds(s * LANES, LANES)
            nyc[sl] = (nyc[sl] - 1900.0) * 0.01

        xbufs = (xb0, xb1)
        trbufs = (tr0, tr1)
        sxs = (sx0, sx1)
        sgs = (sg0, sg1)
        sos = (so0, so1)

        def start_in(c, b):
            t0 = base + c * CHUNK
            cx = pltpu.async_copy(x_hbm.at[pl.ds(t0, CHUNK)], xbufs[b], sxs[b])
            cg = pltpu.async_copy(tab_hbm.at[gic.at[pl.ds(c * CHUNK, CHUNK)]],
                                  trbufs[b], sgs[b])
            return cx, cg

        cur = start_in(0, 0)
        out_copies = [None, None]
        for c in range(N_CHUNKS):
            b = c & 1
            nb = b ^ 1
            if c + 1 < N_CHUNKS:
                if out_copies[nb] is not None:
                    out_copies[nb].wait()
                nxt = start_in(c + 1, nb)
            cur[0].wait()
            cur[1].wait()
            xb = xbufs[b]
            tr = trbufs[b]

            def body(i, _):
                nyw = nyc[pl.ds(c * CHUNK + i, LANES)]
                ny16 = jnp.full((LANES,), nyw[0], jnp.float32)
                for j in range(SLICES):
                    sl = pl.ds(j * LANES, LANES)
                    v = tr[i, sl]
                    if j < HSLICES:
                        v = v + ny16 * wp_v[sl]
                    plsc.addupdate(xb.at[i, sl], v)
                return 0

            lax.fori_loop(0, CHUNK, body, 0)
            out_copies[b] = pltpu.async_copy(
                xb, out_hbm.at[pl.ds(base + c * CHUNK, CHUNK)], sos[b])
            if c + 1 < N_CHUNKS:
                cur = nxt
        out_copies[0].wait()
        out_copies[1].wait()

    return k(xf, gi, by, tab, wp)


def kernel(x, generation_info, birth_years, gen_table, temporal_W, temporal_b):
    B, L, d = x.shape
    xf = x.reshape(B * L, d)
    gi = generation_info.reshape(-1).astype(jnp.int32)
    by = birth_years.reshape(-1)
    bp = jnp.pad(temporal_b, (0, d - temporal_b.shape[0]))
    tab = gen_table + bp[None, :]
    wp = temporal_W[:, 0]
    out = _sc_encode(xf, gi, by, tab, wp)
    return out.reshape(B, L, d)


# speedup vs baseline: 1.4407x; 1.1415x over previous
import functools

import jax
import jax.numpy as jnp
from jax import lax
from jax.experimental import pallas as pl
from jax.experimental.pallas import tpu as pltpu
from jax.experimental.pallas import tpu_sc as plsc

D = 1024
HALF = 512
N_TOKENS = 8192
NW = 32
TPW = N_TOKENS // NW
CHUNK = 8
N_CHUNKS = TPW // CHUNK
NBUF = 4
N_GROUPS = N_CHUNKS // NBUF
LANES = 16
SLICES = D // LANES
HSLICES = HALF // LANES


def _sc_encode(xf, gi, by, tab, wp):
    mesh = plsc.VectorSubcoreMesh(core_axis_name="c", subcore_axis_name="s")

    @functools.partial(
        pl.kernel,
        mesh=mesh,
        out_type=jax.ShapeDtypeStruct((N_TOKENS, D), jnp.float32),
        scratch_types=[
            pltpu.VMEM((HALF,), jnp.float32),
            pltpu.VMEM((TPW,), jnp.int32),
            pltpu.VMEM((TPW + LANES,), jnp.float32),
            pltpu.VMEM((NBUF, CHUNK, D), jnp.float32),
            pltpu.VMEM((NBUF, CHUNK, D), jnp.float32),
            pltpu.SemaphoreType.DMA((NBUF,)),
            pltpu.SemaphoreType.DMA((NBUF,)),
            pltpu.SemaphoreType.DMA((NBUF,)),
        ],
    )
    def k(x_hbm, gi_hbm, by_hbm, tab_hbm, wp_hbm, out_hbm,
          wp_v, gic, nyc, xr, tr, sx, sg, so):
        wid = lax.axis_index("s") * 2 + lax.axis_index("c")
        base = wid * TPW
        pltpu.sync_copy(wp_hbm, wp_v)
        pltpu.sync_copy(gi_hbm.at[pl.ds(base, TPW)], gic)
        pltpu.sync_copy(by_hbm.at[pl.ds(base, TPW)], nyc.at[pl.ds(0, TPW)])
        for s in range(TPW // LANES):
            sl = pl.ds(s * LANES, LANES)
            nyc[sl] = (nyc[sl] - 1900.0) * 0.01

        def issue_in(c, b):
            pltpu.async_copy(x_hbm.at[pl.ds(base + c * CHUNK, CHUNK)],
                             xr.at[b], sx.at[b])
            pltpu.async_copy(tab_hbm.at[gic.at[pl.ds(c * CHUNK, CHUNK)]],
                             tr.at[b], sg.at[b])

        issue_in(0, 0)
        issue_in(1, 1)

        def group(g, _):
            for b in range(NBUF):
                c = g * NBUF + b
                pltpu.make_async_copy(x_hbm.at[pl.ds(base, CHUNK)],
                                      xr.at[b], sx.at[b]).wait()
                pltpu.make_async_copy(x_hbm.at[pl.ds(base, CHUNK)],
                                      tr.at[b], sg.at[b]).wait()

                ny16s = []
                for i in range(CHUNK):
                    nyw = nyc[pl.ds(c * CHUNK + i, LANES)]
                    ny16s.append(jnp.full((LANES,), nyw[0], jnp.float32))

                @plsc.parallel_loop(0, HSLICES, unroll=2)
                def _lo(j):
                    sl = pl.ds(j * LANES, LANES)
                    w = wp_v[sl]
                    for i in range(CHUNK):
                        plsc.addupdate(xr.at[b, i, sl],
                                       tr[b, i, sl] + ny16s[i] * w)

                @plsc.parallel_loop(HSLICES, SLICES, unroll=2)
                def _hi(j):
                    sl = pl.ds(j * LANES, LANES)
                    for i in range(CHUNK):
                        plsc.addupdate(xr.at[b, i, sl], tr[b, i, sl])

                pltpu.async_copy(xr.at[b],
                                 out_hbm.at[pl.ds(base + c * CHUNK, CHUNK)],
                                 so.at[b])
                pb = (b + 2) % NBUF
                cp = c + 2

                @pl.when(cp < N_CHUNKS)
                def _():
                    @pl.when(c >= 2)
                    def _():
                        pltpu.make_async_copy(
                            xr.at[pb], out_hbm.at[pl.ds(base, CHUNK)],
                            so.at[pb]).wait()
                    issue_in(cp, pb)

            return 0

        lax.fori_loop(0, N_GROUPS, group, 0)
        for b in range(NBUF):
            pltpu.make_async_copy(xr.at[b], out_hbm.at[pl.ds(base, CHUNK)],
                                  so.at[b]).wait()

    return k(xf, gi, by, tab, wp)


def kernel(x, generation_info, birth_years, gen_table, temporal_W, temporal_b):
    B, L, d = x.shape
    xf = x.reshape(B * L, d)
    gi = generation_info.reshape(-1).astype(jnp.int32)
    by = birth_years.reshape(-1)
    bp = jnp.pad(temporal_b, (0, d - temporal_b.shape[0]))
    tab = gen_table + bp[None, :]
    wp = temporal_W[:, 0]
    out = _sc_encode(xf, gi, by, tab, wp)
    return out.reshape(B, L, d)


# speedup vs baseline: 2.6727x; 1.8552x over previous
import functools

import jax
import jax.numpy as jnp
from jax import lax
from jax.experimental import pallas as pl
from jax.experimental.pallas import tpu as pltpu
from jax.experimental.pallas import tpu_sc as plsc

D = 1024
HALF = 512
MAX_GEN = 20
N_TOKENS = 8192
NW = 32
TPW = N_TOKENS // NW
CHUNK = 16
N_CHUNKS = TPW // CHUNK
NBUF = 4
N_GROUPS = N_CHUNKS // NBUF
GRP = 8
LANES = 16
SLICES = D // LANES
HSLICES = HALF // LANES


def _sc_encode(xf, gi, by, tab, wp):
    mesh = plsc.VectorSubcoreMesh(core_axis_name="c", subcore_axis_name="s")

    @functools.partial(
        pl.kernel,
        mesh=mesh,
        out_type=jax.ShapeDtypeStruct((N_TOKENS, D), jnp.float32),
        scratch_types=[
            pltpu.VMEM((MAX_GEN, D), jnp.float32),
            pltpu.VMEM((HALF,), jnp.float32),
            pltpu.VMEM((TPW + LANES,), jnp.int32),
            pltpu.VMEM((TPW + LANES,), jnp.float32),
            pltpu.VMEM((NBUF, CHUNK, D), jnp.float32),
            pltpu.SemaphoreType.DMA((NBUF,)),
            pltpu.SemaphoreType.DMA((NBUF,)),
        ],
    )
    def k(x_hbm, gi_hbm, by_hbm, tab_hbm, wp_hbm, out_hbm,
          tab_v, wp_v, gic, nyc, xr, sx, so):
        wid = lax.axis_index("s") * 2 + lax.axis_index("c")
        base = wid * TPW
        pltpu.sync_copy(tab_hbm, tab_v)
        pltpu.sync_copy(wp_hbm, wp_v)
        pltpu.sync_copy(gi_hbm.at[pl.ds(base, TPW)], gic.at[pl.ds(0, TPW)])
        pltpu.sync_copy(by_hbm.at[pl.ds(base, TPW)], nyc.at[pl.ds(0, TPW)])
        for s in range(TPW // LANES):
            sl = pl.ds(s * LANES, LANES)
            nyc[sl] = (nyc[sl] - 1900.0) * 0.01

        def issue_in(c, b):
            pltpu.async_copy(x_hbm.at[pl.ds(base + c * CHUNK, CHUNK)],
                             xr.at[b], sx.at[b])

        issue_in(0, 0)
        issue_in(1, 1)

        def group(g, _):
            for b in range(NBUF):
                c = g * NBUF + b
                pltpu.make_async_copy(x_hbm.at[pl.ds(base, CHUNK)],
                                      xr.at[b], sx.at[b]).wait()

                for t0 in range(0, CHUNK, GRP):
                    gids = []
                    ny16s = []
                    for i in range(t0, t0 + GRP):
                        gw = gic[pl.ds(c * CHUNK + i, LANES)]
                        gids.append(gw[0])
                        nyw = nyc[pl.ds(c * CHUNK + i, LANES)]
                        ny16s.append(jnp.full((LANES,), nyw[0], jnp.float32))

                    @plsc.parallel_loop(0, HSLICES, unroll=2)
                    def _lo(j):
                        sl = pl.ds(j * LANES, LANES)
                        w = wp_v[sl]
                        for i in range(GRP):
                            plsc.addupdate(
                                xr.at[b, t0 + i, sl],
                                tab_v[gids[i], sl] + ny16s[i] * w)

                    @plsc.parallel_loop(HSLICES, SLICES, unroll=2)
                    def _hi(j):
                        sl = pl.ds(j * LANES, LANES)
                        for i in range(GRP):
                            plsc.addupdate(xr.at[b, t0 + i, sl],
                                           tab_v[gids[i], sl])

                pltpu.async_copy(xr.at[b],
                                 out_hbm.at[pl.ds(base + c * CHUNK, CHUNK)],
                                 so.at[b])
                pb = (b + 2) % NBUF
                cp = c + 2

                @pl.when(cp < N_CHUNKS)
                def _():
                    @pl.when(c >= 2)
                    def _():
                        pltpu.make_async_copy(
                            xr.at[pb], out_hbm.at[pl.ds(base, CHUNK)],
                            so.at[pb]).wait()
                    issue_in(cp, pb)

            return 0

        lax.fori_loop(0, N_GROUPS, group, 0)
        for b in range(NBUF):
            pltpu.make_async_copy(xr.at[b], out_hbm.at[pl.ds(base, CHUNK)],
                                  so.at[b]).wait()

    return k(xf, gi, by, tab, wp)


def kernel(x, generation_info, birth_years, gen_table, temporal_W, temporal_b):
    B, L, d = x.shape
    xf = x.reshape(B * L, d)
    gi = generation_info.reshape(-1).astype(jnp.int32)
    by = birth_years.reshape(-1)
    bp = jnp.pad(temporal_b, (0, d - temporal_b.shape[0]))
    tab = gen_table + bp[None, :]
    wp = temporal_W[:, 0]
    out = _sc_encode(xf, gi, by, tab, wp)
    return out.reshape(B, L, d)


# speedup vs baseline: 2.7340x; 1.0229x over previous
import functools

import jax
import jax.numpy as jnp
from jax import lax
from jax.experimental import pallas as pl
from jax.experimental.pallas import tpu as pltpu
from jax.experimental.pallas import tpu_sc as plsc

D = 1024
HALF = 512
MAX_GEN = 20
N_TOKENS = 8192
NW = 32
TPW = N_TOKENS // NW
CHUNK = 16
N_CHUNKS = TPW // CHUNK
NBUF = 4
N_GROUPS = N_CHUNKS // NBUF
GRP = 8
LANES = 16
SLICES = D // LANES
HSLICES = HALF // LANES


def _sc_encode(xf, gi, by, tab, wp):
    mesh = plsc.VectorSubcoreMesh(core_axis_name="c", subcore_axis_name="s")

    @functools.partial(
        pl.kernel,
        mesh=mesh,
        out_type=jax.ShapeDtypeStruct((N_TOKENS, D), jnp.float32),
        scratch_types=[
            pltpu.VMEM((MAX_GEN, D), jnp.float32),
            pltpu.VMEM((HALF,), jnp.float32),
            pltpu.VMEM((TPW + LANES,), jnp.int32),
            pltpu.VMEM((TPW + LANES,), jnp.float32),
            pltpu.VMEM((NBUF, CHUNK, D), jnp.float32),
            pltpu.SemaphoreType.DMA((NBUF,)),
            pltpu.SemaphoreType.DMA((NBUF,)),
        ],
    )
    def k(x_hbm, gi_hbm, by_hbm, tab_hbm, wp_hbm, out_hbm,
          tab_v, wp_v, gic, nyc, xr, sx, so):
        wid = lax.axis_index("s") * 2 + lax.axis_index("c")
        base = wid * TPW

        def issue_in(c, b):
            pltpu.async_copy(x_hbm.at[pl.ds(base + c * CHUNK, CHUNK)],
                             xr.at[b], sx.at[b])

        issue_in(0, 0)
        issue_in(1, 1)
        stg = [
            pltpu.async_copy(tab_hbm, tab_v, so.at[0]),
            pltpu.async_copy(wp_hbm, wp_v, so.at[1]),
            pltpu.async_copy(gi_hbm.at[pl.ds(base, TPW)],
                             gic.at[pl.ds(0, TPW)], so.at[2]),
            pltpu.async_copy(by_hbm.at[pl.ds(base, TPW)],
                             nyc.at[pl.ds(0, TPW)], so.at[3]),
        ]
        for cp in stg:
            cp.wait()
        for s in range(TPW // LANES):
            sl = pl.ds(s * LANES, LANES)
            nyc[sl] = (nyc[sl] - 1900.0) * 0.01

        def group(g, _):
            for b in range(NBUF):
                c = g * NBUF + b
                pltpu.make_async_copy(x_hbm.at[pl.ds(base, CHUNK)],
                                      xr.at[b], sx.at[b]).wait()

                for t0 in range(0, CHUNK, GRP):
                    gids = []
                    ny16s = []
                    for i in range(t0, t0 + GRP):
                        gw = gic[pl.ds(c * CHUNK + i, LANES)]
                        gids.append(gw[0])
                        nyw = nyc[pl.ds(c * CHUNK + i, LANES)]
                        ny16s.append(jnp.full((LANES,), nyw[0], jnp.float32))

                    @plsc.parallel_loop(0, HSLICES, unroll=2)
                    def _lo(j):
                        sl = pl.ds(j * LANES, LANES)
                        w = wp_v[sl]
                        for i in range(GRP):
                            plsc.addupdate(
                                xr.at[b, t0 + i, sl],
                                tab_v[gids[i], sl] + ny16s[i] * w)

                    @plsc.parallel_loop(HSLICES, SLICES, unroll=2)
                    def _hi(j):
                        sl = pl.ds(j * LANES, LANES)
                        for i in range(GRP):
                            plsc.addupdate(xr.at[b, t0 + i, sl],
                                           tab_v[gids[i], sl])

                pltpu.async_copy(xr.at[b],
                                 out_hbm.at[pl.ds(base + c * CHUNK, CHUNK)],
                                 so.at[b])
                pb = (b + 2) % NBUF
                cp = c + 2

                @pl.when(cp < N_CHUNKS)
                def _():
                    @pl.when(c >= 2)
                    def _():
                        pltpu.make_async_copy(
                            xr.at[pb], out_hbm.at[pl.ds(base, CHUNK)],
                            so.at[pb]).wait()
                    issue_in(cp, pb)

            return 0

        lax.fori_loop(0, N_GROUPS, group, 0)
        for b in range(NBUF):
            pltpu.make_async_copy(xr.at[b], out_hbm.at[pl.ds(base, CHUNK)],
                                  so.at[b]).wait()

    return k(xf, gi, by, tab, wp)


def kernel(x, generation_info, birth_years, gen_table, temporal_W, temporal_b):
    B, L, d = x.shape
    xf = x.reshape(B * L, d)
    gi = generation_info.reshape(-1).astype(jnp.int32)
    by = birth_years.reshape(-1)
    bp = jnp.pad(temporal_b, (0, d - temporal_b.shape[0]))
    tab = gen_table + bp[None, :]
    wp = temporal_W[:, 0]
    out = _sc_encode(xf, gi, by, tab, wp)
    return out.reshape(B, L, d)
